# baseline (speedup 1.0000x reference)
import jax
import jax.numpy as jnp
from jax import lax
from jax.experimental import pallas as pl
from jax.experimental.pallas import tpu as pltpu
from jax.experimental.pallas import tpu_sc as plsc

_ROWS = 16384
_COLS = 1000
_NC = 2
_NS = 16
_NW = _NC * _NS
_RPW = _ROWS // _NW
_RPB = 16
_BPW = _RPW // _RPB
_BLKW = _RPB * _COLS


def _sc_body(x_hbm, out_hbm, in_v, out_v):
    wid = lax.axis_index("s") * _NC + lax.axis_index("c")
    base_vec = lax.iota(jnp.int32, 16) * _COLS

    def zbody(i, carry):
        out_v[pl.ds(i * 16, 16)] = jnp.zeros((16,), jnp.float32)
        return carry

    lax.fori_loop(0, _BLKW // 16, zbody, 0)

    def block_body(b, carry):
        off = (wid * _BPW + b) * _BLKW
        pltpu.sync_copy(x_hbm.at[pl.ds(off, _BLKW)], in_v)

        def scan_body(c, mi):
            m, idx = mi
            v = plsc.load_gather(in_v, [base_vec + c])
            upd = v > m
            m = jnp.where(upd, v, m)
            idx = jnp.where(upd, jnp.full((16,), c, jnp.int32), idx)
            return m, idx

        m0 = jnp.full((16,), -jnp.inf, jnp.float32)
        i0 = jnp.zeros((16,), jnp.int32)
        _, idx = lax.fori_loop(0, _COLS, scan_body, (m0, i0), unroll=8)

        pos = base_vec + idx
        plsc.store_scatter(out_v, [pos], jnp.ones((16,), jnp.float32))
        pltpu.sync_copy(out_v, out_hbm.at[pl.ds(off, _BLKW)])
        plsc.store_scatter(out_v, [pos], jnp.zeros((16,), jnp.float32))
        return carry

    lax.fori_loop(0, _BPW, block_body, 0)


_sc_call = pl.kernel(
    _sc_body,
    out_type=jax.ShapeDtypeStruct((_ROWS * _COLS,), jnp.float32),
    mesh=plsc.VectorSubcoreMesh(
        core_axis_name="c", subcore_axis_name="s",
        num_cores=_NC, num_subcores=_NS),
    scratch_types=[
        pltpu.VMEM((_BLKW,), jnp.float32),
        pltpu.VMEM((_BLKW,), jnp.float32),
    ],
    compiler_params=pltpu.CompilerParams(needs_layout_passes=False),
)


def kernel(x):
    return _sc_call(x.reshape(-1)).reshape(_ROWS, _COLS)


# speedup vs baseline: 1.2007x; 1.2007x over previous
import jax
import jax.numpy as jnp
from jax import lax
from jax.experimental import pallas as pl
from jax.experimental.pallas import tpu as pltpu
from jax.experimental.pallas import tpu_sc as plsc

_ROWS = 16384
_COLS = 1000
_NC = 2
_NS = 16
_NW = _NC * _NS
_RPW = _ROWS // _NW
_RPB = 16
_BPW = _RPW // _RPB
_BLKW = _RPB * _COLS
_NCH = _COLS // 16
_TAIL = _COLS - 16
_BIG = 1 << 30


def _sc_body(x_hbm, out_hbm, in0, in1, o0, o1, mt, ct, p0, p1,
             si0, si1, so0, so1):
    ins, outs, poss = (in0, in1), (o0, o1), (p0, p1)
    sis, sos = (si0, si1), (so0, so1)
    wid = lax.axis_index("s") * _NC + lax.axis_index("c")
    lane = lax.iota(jnp.int32, 16)
    base = wid * _RPW * _COLS

    def in_copy(b, k):
        return pltpu.make_async_copy(
            x_hbm.at[pl.ds(base + b * _BLKW, _BLKW)], ins[k], sis[k])

    def out_copy(b, k):
        return pltpu.make_async_copy(
            outs[k], out_hbm.at[pl.ds(base + b * _BLKW, _BLKW)], sos[k])

    def zbody(i, carry):
        o0[pl.ds(i * 16, 16)] = jnp.zeros((16,), jnp.float32)
        o1[pl.ds(i * 16, 16)] = jnp.zeros((16,), jnp.float32)
        return carry

    lax.fori_loop(0, _BLKW // 16, zbody, 0)
    p0[...] = jnp.zeros((16,), jnp.int32)
    p1[...] = jnp.zeros((16,), jnp.int32)

    in_copy(0, 0).start()
    in_copy(1, 1).start()

    def group(g, carry):
        for k in range(2):
            b = g * 2 + k
            in_copy(b, k).wait()

            def row_body(r, carry2):
                rbase = r * _COLS

                def cbody(c, mi):
                    m, cb, soff = mi
                    v = ins[k][pl.ds(rbase + c * 16, 16)]
                    upd = v > m
                    m = jnp.maximum(m, v)
                    cb = jnp.where(upd, soff, cb)
                    return m, cb, soff + 16

                m0 = jnp.full((16,), -jnp.inf, jnp.float32)
                z = jnp.zeros((16,), jnp.int32)
                m, cb, _ = lax.fori_loop(
                    0, _NCH, cbody, (m0, z, z), unroll=8)
                v = ins[k][pl.ds(rbase + _TAIL, 16)]
                upd = v > m
                m = jnp.maximum(m, v)
                cb = jnp.where(upd, jnp.full((16,), _TAIL, jnp.int32), cb)
                tidx = lane * 17 + r
                plsc.store_scatter(mt, [tidx], m)
                plsc.store_scatter(ct, [tidx], cb)
                return carry2

            lax.fori_loop(0, _RPB, row_body, 0)

            gm = jnp.full((16,), -jnp.inf, jnp.float32)
            for l in range(16):
                gm = jnp.maximum(gm, plsc.load_gather(mt, [lane + 17 * l]))
            best = jnp.full((16,), _BIG, jnp.int32)
            for l in range(16):
                vl = plsc.load_gather(mt, [lane + 17 * l])
                cl = plsc.load_gather(ct, [lane + 17 * l])
                cand = jnp.where(vl == gm, cl + l, _BIG)
                best = jnp.minimum(best, cand)

            @pl.when(b + 2 < _BPW)
            def _():
                in_copy(b + 2, k).start()

            @pl.when(b >= 2)
            def _():
                out_copy(b - 2, k).wait()

            oldpos = poss[k][...]
            plsc.store_scatter(outs[k], [oldpos],
                               jnp.zeros((16,), jnp.float32))
            pos = lane * _COLS + best
            plsc.store_scatter(outs[k], [pos], jnp.ones((16,), jnp.float32))
            poss[k][...] = pos
            out_copy(b, k).start()
        return carry

    lax.fori_loop(0, _BPW // 2, group, 0)
    out_copy(_BPW - 2, 0).wait()
    out_copy(_BPW - 1, 1).wait()


_sc_call = pl.kernel(
    _sc_body,
    out_type=jax.ShapeDtypeStruct((_ROWS * _COLS,), jnp.float32),
    mesh=plsc.VectorSubcoreMesh(
        core_axis_name="c", subcore_axis_name="s",
        num_cores=_NC, num_subcores=_NS),
    scratch_types=[
        pltpu.VMEM((_BLKW,), jnp.float32),
        pltpu.VMEM((_BLKW,), jnp.float32),
        pltpu.VMEM((_BLKW,), jnp.float32),
        pltpu.VMEM((_BLKW,), jnp.float32),
        pltpu.VMEM((16 * 17,), jnp.float32),
        pltpu.VMEM((16 * 17,), jnp.int32),
        pltpu.VMEM((16,), jnp.int32),
        pltpu.VMEM((16,), jnp.int32),
        pltpu.SemaphoreType.DMA,
        pltpu.SemaphoreType.DMA,
        pltpu.SemaphoreType.DMA,
        pltpu.SemaphoreType.DMA,
    ],
    compiler_params=pltpu.CompilerParams(needs_layout_passes=False),
)


def kernel(x):
    return _sc_call(x.reshape(-1)).reshape(_ROWS, _COLS)


# speedup vs baseline: 2.0630x; 1.7183x over previous
import jax
import jax.numpy as jnp
from jax import lax
from jax.experimental import pallas as pl
from jax.experimental.pallas import tpu as pltpu
from jax.experimental.pallas import tpu_sc as plsc

_ROWS = 16384
_COLS = 1000
_NC = 2
_NS = 16
_NW = _NC * _NS
_RPW = _ROWS // _NW
_RPB = 16
_BPW = _RPW // _RPB
_NCH = _COLS // 16
_TAIL = _COLS - 16
_BIG = 1 << 30


def _sc_body(x_hbm, out_hbm, in0, in1, o0, o1, mt, ct, p0, p1,
             si0, si1, so0, so1):
    ins, outs, poss = (in0, in1), (o0, o1), (p0, p1)
    sis, sos = (si0, si1), (so0, so1)
    wid = lax.axis_index("s") * _NC + lax.axis_index("c")
    lane = lax.iota(jnp.int32, 16)
    row0 = wid * _RPW

    def in_copy(b, k):
        return pltpu.make_async_copy(
            x_hbm.at[pl.ds(row0 + b * _RPB, _RPB), :], ins[k], sis[k])

    def out_copy(b, k):
        return pltpu.make_async_copy(
            outs[k], out_hbm.at[pl.ds(row0 + b * _RPB, _RPB), :], sos[k])

    def zrow(r, carry):
        def zchunk(i, c2):
            o0[r, pl.ds(i * 16, 16)] = jnp.zeros((16,), jnp.float32)
            o1[r, pl.ds(i * 16, 16)] = jnp.zeros((16,), jnp.float32)
            return c2

        lax.fori_loop(0, _NCH, zchunk, 0)
        o0[r, pl.ds(_TAIL, 16)] = jnp.zeros((16,), jnp.float32)
        o1[r, pl.ds(_TAIL, 16)] = jnp.zeros((16,), jnp.float32)
        return carry

    lax.fori_loop(0, _RPB, zrow, 0)
    p0[...] = jnp.zeros((16,), jnp.int32)
    p1[...] = jnp.zeros((16,), jnp.int32)

    in_copy(0, 0).start()
    in_copy(1, 1).start()

    def group(g, carry):
        for k in range(2):
            b = g * 2 + k
            in_copy(b, k).wait()

            def row_body(r, carry2):
                def cbody(c, mi):
                    m, cb, soff = mi
                    v = ins[k][r, pl.ds(c * 16, 16)]
                    upd = v > m
                    m = jnp.maximum(m, v)
                    cb = jnp.where(upd, soff, cb)
                    return m, cb, soff + 16

                m0 = jnp.full((16,), -jnp.inf, jnp.float32)
                z = jnp.zeros((16,), jnp.int32)
                m, cb, _ = lax.fori_loop(
                    0, _NCH, cbody, (m0, z, z), unroll=8)
                v = ins[k][r, pl.ds(_TAIL, 16)]
                upd = v > m
                m = jnp.maximum(m, v)
                cb = jnp.where(upd, jnp.full((16,), _TAIL, jnp.int32), cb)
                tidx = lane * 17 + r
                plsc.store_scatter(mt, [tidx], m)
                plsc.store_scatter(ct, [tidx], cb)
                return carry2

            lax.fori_loop(0, _RPB, row_body, 0)

            gm = jnp.full((16,), -jnp.inf, jnp.float32)
            for l in range(16):
                gm = jnp.maximum(gm, plsc.load_gather(mt, [lane + 17 * l]))
            best = jnp.full((16,), _BIG, jnp.int32)
            for l in range(16):
                vl = plsc.load_gather(mt, [lane + 17 * l])
                cl = plsc.load_gather(ct, [lane + 17 * l])
                cand = jnp.where(vl == gm, cl + l, _BIG)
                best = jnp.minimum(best, cand)

            @pl.when(b + 2 < _BPW)
            def _():
                in_copy(b + 2, k).start()

            @pl.when(b >= 2)
            def _():
                out_copy(b - 2, k).wait()

            oldbest = poss[k][...]
            plsc.store_scatter(outs[k], [lane, oldbest],
                               jnp.zeros((16,), jnp.float32))
            plsc.store_scatter(outs[k], [lane, best],
                               jnp.ones((16,), jnp.float32))
            poss[k][...] = best
            out_copy(b, k).start()
        return carry

    lax.fori_loop(0, _BPW // 2, group, 0)
    out_copy(_BPW - 2, 0).wait()
    out_copy(_BPW - 1, 1).wait()


_sc_call = pl.kernel(
    _sc_body,
    out_type=jax.ShapeDtypeStruct((_ROWS, _COLS), jnp.float32),
    mesh=plsc.VectorSubcoreMesh(
        core_axis_name="c", subcore_axis_name="s",
        num_cores=_NC, num_subcores=_NS),
    scratch_types=[
        pltpu.VMEM((_RPB, _COLS), jnp.float32),
        pltpu.VMEM((_RPB, _COLS), jnp.float32),
        pltpu.VMEM((_RPB, _COLS), jnp.float32),
        pltpu.VMEM((_RPB, _COLS), jnp.float32),
        pltpu.VMEM((16 * 17,), jnp.float32),
        pltpu.VMEM((16 * 17,), jnp.int32),
        pltpu.VMEM((16,), jnp.int32),
        pltpu.VMEM((16,), jnp.int32),
        pltpu.SemaphoreType.DMA,
        pltpu.SemaphoreType.DMA,
        pltpu.SemaphoreType.DMA,
        pltpu.SemaphoreType.DMA,
    ],
    compiler_params=pltpu.CompilerParams(needs_layout_passes=False),
)


def kernel(x):
    return _sc_call(x)
